# Initial kernel scaffold; baseline (speedup 1.0000x reference)
import jax
import jax.numpy as jnp
from jax.experimental import pallas as pl


def kernel(x, edge_index, batch, W1, b1, W2, b2, fw1, fb1, fw2, fb2):
    raise NotImplementedError("write your pallas kernel here")


# speedup vs baseline: 12.9939x; 12.9939x over previous
import functools
import jax
import jax.numpy as jnp
from jax import lax
from jax.experimental import pallas as pl
from jax.experimental.pallas import tpu as pltpu
from jax.experimental.pallas import tpu_sc as plsc

N = 10000
E = 320000
D = 128
G = 64
NC = 2
NS = 16
NW = NC * NS
NP = 10240
RT = NP // NW
RZ = NP // NS
K = 128
CPT = -(-(E // NW) // K)
EPW = CPT * K
EP = NW * EPW
CH = 160

_mesh = plsc.VectorSubcoreMesh(core_axis_name="c", subcore_axis_name="s")


@functools.partial(
    pl.kernel,
    out_type=jax.ShapeDtypeStruct((NW, NP), jnp.float32),
    mesh=_mesh,
    scratch_types=[
        pltpu.VMEM((CPT, K), jnp.int32),
        pltpu.VMEM((NP,), jnp.float32),
    ],
    compiler_params=pltpu.CompilerParams(needs_layout_passes=False),
)
def _sc_degree(dst2d, zeros1d, out, dst_v, acc):
  cid = lax.axis_index("c")
  sid = lax.axis_index("s")
  w = cid * NS + sid
  pltpu.sync_copy(zeros1d, acc)
  pltpu.sync_copy(dst2d.at[w], dst_v)
  ones16 = jnp.ones((16,), jnp.float32)

  @pl.loop(0, CPT)
  def _(j):
    for c in range(8):
      idx = dst_v[j, pl.ds(c * 16, 16)]
      plsc.addupdate_scatter(acc, [idx], ones16)

  pltpu.sync_copy(acc, out.at[w])


@functools.partial(
    pl.kernel,
    out_type=jax.ShapeDtypeStruct((NC, NP, D), jnp.float32),
    mesh=_mesh,
    scratch_types=[
        pltpu.VMEM((CPT, K), jnp.int32),
        pltpu.VMEM((CPT, K), jnp.int32),
        pltpu.VMEM((K, D), jnp.float32),
        pltpu.VMEM_SHARED((NP, D), jnp.float32),
        pltpu.SemaphoreType.DMA,
    ],
)
def _sc_scatter(src2d, dst2d, y, zrows, out, src_v, dst_v, rows_v, acc, sem):
  cid = lax.axis_index("c")
  sid = lax.axis_index("s")
  w = cid * NS + sid
  pltpu.sync_copy(zrows, acc.at[pl.ds(sid * RZ, RZ)])
  pltpu.sync_copy(src2d.at[w], src_v)
  pltpu.sync_copy(dst2d.at[w], dst_v)
  plsc.subcore_barrier()

  @pl.loop(0, CPT)
  def _(j):
    pltpu.async_copy(y.at[src_v.at[j]], rows_v, sem).wait()
    pltpu.sync_copy(rows_v, acc.at[dst_v.at[j]], add=True)

  plsc.subcore_barrier()
  pltpu.sync_copy(acc.at[pl.ds(sid * RZ, RZ)],
                  out.at[cid, pl.ds(sid * RZ, RZ)])


@functools.partial(
    pl.kernel,
    out_type=jax.ShapeDtypeStruct((NW, G * D), jnp.float32),
    mesh=_mesh,
    scratch_types=[
        pltpu.VMEM((RT * D,), jnp.float32),
        pltpu.VMEM((G * D,), jnp.float32),
        pltpu.VMEM((RT,), jnp.int32),
    ],
    compiler_params=pltpu.CompilerParams(needs_layout_passes=False),
)
def _sc_segmax(h2, batch, zacc, out, h2_v, acc_v, batch_v):
  cid = lax.axis_index("c")
  sid = lax.axis_index("s")
  w = cid * NS + sid
  base = w * RT
  pltpu.sync_copy(zacc, acc_v)
  pltpu.sync_copy(h2.at[pl.ds(base * D, RT * D)], h2_v)
  pltpu.sync_copy(batch.at[pl.ds(base, RT)], batch_v)

  nrows = jnp.minimum(RT, jnp.maximum(N - base, 0))
  iotas = [lax.iota(jnp.int32, 16) + c * 16 for c in range(8)]

  @pl.loop(0, nrows)
  def _(i):
    gidv = plsc.load_gather(batch_v, [jnp.full((16,), 0, jnp.int32) + i])
    gbase = gidv * D
    for c in range(8):
      h = h2_v[pl.ds(i * D + c * 16, 16)]
      idx = gbase + iotas[c]
      g = plsc.load_gather(acc_v, [idx])
      plsc.store_scatter(acc_v, [idx], jnp.maximum(g, h))

  pltpu.sync_copy(acc_v, out.at[w])


def _tc_xw_body(x_ref, w_ref, hist_ref, y_ref, dinv_ref):
  deg = 1.0 + jnp.sum(hist_ref[...], axis=0)
  dinv = lax.rsqrt(jnp.maximum(deg, 1.0))
  y = jnp.dot(x_ref[...], w_ref[...], preferred_element_type=jnp.float32)
  y_ref[...] = y * dinv[:, None]
  dinv_ref[...] = dinv


def _tc_mid_body(p0_ref, p1_ref, y1_ref, dinv_ref, b_ref, w_ref, y2_ref):
  dinv = dinv_ref[...]
  t = (p0_ref[...] + p1_ref[...] + y1_ref[...]) * dinv[:, None] + b_ref[...]
  h = jnp.maximum(t, 0.0)
  y2 = jnp.dot(h, w_ref[...], preferred_element_type=jnp.float32)
  y2_ref[...] = y2 * dinv[:, None]


def _tc_h2_body(p0_ref, p1_ref, y2_ref, dinv_ref, b_ref, h2_ref):
  t = ((p0_ref[...] + p1_ref[...] + y2_ref[...]) * dinv_ref[...][:, None]
       + b_ref[...])
  h2_ref[...] = jnp.maximum(t, 0.0)


def _tc_head_body(gm_ref, fw1_ref, fb1_ref, fw2_ref, fb2_ref, out_ref):
  g = jnp.max(gm_ref[...], axis=0)
  z = jnp.maximum(
      jnp.dot(g, fw1_ref[...], preferred_element_type=jnp.float32)
      + fb1_ref[...], 0.0)
  logits = (jnp.dot(z, fw2_ref[...], preferred_element_type=jnp.float32)
            + fb2_ref[...])
  col = lax.broadcasted_iota(jnp.int32, logits.shape, 1)
  valid = col < 2
  neg = jnp.float32(-3.0e38)
  masked = jnp.where(valid, logits, neg)
  m = jnp.max(masked, axis=1, keepdims=True)
  s = jnp.sum(jnp.where(valid, jnp.exp(masked - m), 0.0), axis=1,
              keepdims=True)
  out_ref[...] = logits - (m + jnp.log(s))


BR = 2048


def _tc_xw(x, w, hist):
  grid = (NP // BR,)
  return pl.pallas_call(
      _tc_xw_body,
      grid=grid,
      in_specs=[
          pl.BlockSpec((BR, D), lambda i: (i, 0)),
          pl.BlockSpec((D, D), lambda i: (0, 0)),
          pl.BlockSpec((NW, BR), lambda i: (0, i)),
      ],
      out_specs=[
          pl.BlockSpec((BR, D), lambda i: (i, 0)),
          pl.BlockSpec((BR,), lambda i: (i,)),
      ],
      out_shape=[
          jax.ShapeDtypeStruct((NP, D), jnp.float32),
          jax.ShapeDtypeStruct((NP,), jnp.float32),
      ],
  )(x, w, hist)


def _tc_mid(p0, p1, y1, dinv, b, w):
  grid = (NP // BR,)
  return pl.pallas_call(
      _tc_mid_body,
      grid=grid,
      in_specs=[
          pl.BlockSpec((BR, D), lambda i: (i, 0)),
          pl.BlockSpec((BR, D), lambda i: (i, 0)),
          pl.BlockSpec((BR, D), lambda i: (i, 0)),
          pl.BlockSpec((BR,), lambda i: (i,)),
          pl.BlockSpec((D,), lambda i: (0,)),
          pl.BlockSpec((D, D), lambda i: (0, 0)),
      ],
      out_specs=pl.BlockSpec((BR, D), lambda i: (i, 0)),
      out_shape=jax.ShapeDtypeStruct((NP, D), jnp.float32),
  )(p0, p1, y1, dinv, b, w)


def _tc_h2(p0, p1, y2, dinv, b):
  grid = (NP // BR,)
  return pl.pallas_call(
      _tc_h2_body,
      grid=grid,
      in_specs=[
          pl.BlockSpec((BR, D), lambda i: (i, 0)),
          pl.BlockSpec((BR, D), lambda i: (i, 0)),
          pl.BlockSpec((BR, D), lambda i: (i, 0)),
          pl.BlockSpec((BR,), lambda i: (i,)),
          pl.BlockSpec((D,), lambda i: (0,)),
      ],
      out_specs=pl.BlockSpec((BR, D), lambda i: (i, 0)),
      out_shape=jax.ShapeDtypeStruct((NP, D), jnp.float32),
  )(p0, p1, y2, dinv, b)


def _tc_head(gm, fw1, fb1, fw2p, fb2p):
  return pl.pallas_call(
      _tc_head_body,
      out_shape=jax.ShapeDtypeStruct((G, D), jnp.float32),
  )(gm, fw1, fb1, fw2p, fb2p)


def kernel(x, edge_index, batch, W1, b1, W2, b2, fw1, fb1, fw2, fb2):
  f32 = jnp.float32
  x_pad = jnp.zeros((NP, D), f32).at[:N].set(x)
  src = jnp.full((EP,), N, jnp.int32).at[:E].set(edge_index[0].astype(jnp.int32))
  dst = jnp.full((EP,), N, jnp.int32).at[:E].set(edge_index[1].astype(jnp.int32))
  src2d = src.reshape(NW, CPT, K)
  dst2d = dst.reshape(NW, CPT, K)
  batch_pad = jnp.zeros((NP,), jnp.int32).at[:N].set(batch.astype(jnp.int32))

  zeros1d = jnp.zeros((NP,), f32)
  zrows = jnp.zeros((RZ, D), f32)
  zacc = jnp.zeros((G * D,), f32)

  hist = _sc_degree(dst2d, zeros1d)

  y1, dinv = _tc_xw(x_pad, W1, hist)
  p = _sc_scatter(src2d, dst2d, y1, zrows)
  y2 = _tc_mid(p[0], p[1], y1, dinv, b1, W2)
  p2 = _sc_scatter(src2d, dst2d, y2, zrows)
  h2 = _tc_h2(p2[0], p2[1], y2, dinv, b2)
  gm = _sc_segmax(h2.reshape(NP * D), batch_pad, zacc)
  out = _tc_head(gm.reshape(NW, G, D), fw1, fb1,
                 jnp.zeros((D, D), f32).at[:, :2].set(fw2),
                 jnp.zeros((D,), f32).at[:2].set(fb2))
  return out[:, :2]
